# baseline (speedup 1.0000x reference)
import jax
import jax.numpy as jnp
from jax.experimental import pallas as pl
from jax.experimental.pallas import tpu as pltpu

P = 4096
B = 6
F_IN = 960
NCOL = 12
G = 16
NP = B * G
PB1 = 1024
NP1 = P // PB1
NS1 = B * NP1
PBLK = 512
NPB = P // PBLK


def _fused_kernel(emb_ref, refc_ref, wsh_ref, wref_ref, adj_ref, bias_ref,
                  pm_ref, rgb_ref, tpk_scr):
    i = pl.program_id(0)

    @pl.when(i < NS1)
    def _project():
        part = jnp.dot(
            emb_ref[0].astype(jnp.bfloat16), wsh_ref[0],
            preferred_element_type=jnp.float32,
        )
        row = (i % NP1) * PB1

        @pl.when(i < NP1)
        def _first_view():
            rw = jnp.dot(refc_ref[...], wref_ref[...],
                         preferred_element_type=jnp.float32)
            tpk_scr[pl.ds(row, PB1), :] = part + rw

        @pl.when(i >= NP1)
        def _accum_views():
            tpk_scr[pl.ds(row, PB1), :] = tpk_scr[pl.ds(row, PB1), :] + part

    @pl.when(i >= NS1)
    def _aggregate():
        acc = jnp.dot(adj_ref[...], tpk_scr[...], preferred_element_type=jnp.float32)
        tp = tpk_scr[pl.ds((i - NS1) * PBLK, PBLK), :]
        bz = bias_ref[...]
        for b in range(B):
            g = b * G
            pm_ref[b] = jnp.tanh(acc[:, g:g + 3] + tp[:, g + 6:g + 9] + bz[:, g:g + 3])
            rgb_ref[b] = jax.nn.sigmoid(
                acc[:, g + 3:g + 6] + tp[:, g + 9:g + 12] + bz[:, g + 3:g + 6]
            )


def kernel(embeddings, ref, adj, W_d, Wl_d, b_d, W_r, Wl_r, b_r):
    refc = ref.reshape(P, 3)
    W_all = jnp.concatenate([W_d, W_r, Wl_d, Wl_r], axis=1)
    W_emb = W_all[:F_IN]
    W_ref = W_all[F_IN:]
    w_sh = jnp.stack([
        jnp.concatenate(
            [jnp.zeros((F_IN, b * G), jnp.float32), W_emb,
             jnp.zeros((F_IN, NP - b * G - NCOL), jnp.float32)], axis=1)
        for b in range(B)
    ]).astype(jnp.bfloat16)
    wref_pk = jnp.tile(
        jnp.concatenate([W_ref, jnp.zeros((3, G - NCOL), jnp.float32)], axis=1),
        (1, B),
    )
    bias = jnp.tile(
        jnp.concatenate([b_d, b_r, jnp.zeros((G - 6,), jnp.float32)]), B
    ).reshape(1, NP)

    pm, rgb = pl.pallas_call(
        _fused_kernel,
        grid=(NS1 + NPB,),
        in_specs=[
            pl.BlockSpec((1, PB1, F_IN),
                         lambda i: (jnp.minimum(i, NS1 - 1) // NP1,
                                    jnp.minimum(i, NS1 - 1) % NP1, 0)),
            pl.BlockSpec((PB1, 3),
                         lambda i: (jnp.minimum(i, NS1 - 1) % NP1, 0)),
            pl.BlockSpec((1, F_IN, NP),
                         lambda i: (jnp.minimum(i, NS1 - 1) // NP1, 0, 0)),
            pl.BlockSpec((3, NP), lambda i: (0, 0)),
            pl.BlockSpec((PBLK, P),
                         lambda i: (jnp.clip(i - NS1, 0, NPB - 1), 0)),
            pl.BlockSpec((1, NP), lambda i: (0, 0)),
        ],
        out_specs=[
            pl.BlockSpec((B, PBLK, 3), lambda i: (0, jnp.maximum(i - NS1, 0), 0)),
            pl.BlockSpec((B, PBLK, 3), lambda i: (0, jnp.maximum(i - NS1, 0), 0)),
        ],
        out_shape=[
            jax.ShapeDtypeStruct((B, P, 3), jnp.float32),
            jax.ShapeDtypeStruct((B, P, 3), jnp.float32),
        ],
        scratch_shapes=[pltpu.VMEM((P, NP), jnp.float32)],
        compiler_params=pltpu.CompilerParams(
            dimension_semantics=("arbitrary",),
        ),
    )(embeddings, refc, w_sh, wref_pk, adj, bias)
    return pm, rgb


# speedup vs baseline: 1.0481x; 1.0481x over previous
import jax
import jax.numpy as jnp
from jax.experimental import pallas as pl
from jax.experimental.pallas import tpu as pltpu

P = 4096
B = 6
F_IN = 960
NCOL = 12
G = 16
NP = B * G
PB1 = 512
NP1 = P // PB1
PBLK = 512
NPB = P // PBLK


def _fused_kernel(emb_ref, refc_ref, w_emb_ref, w_ref_ref, adj_ref, bias_ref,
                  pm_ref, rgb_ref, tpk_scr):
    i = pl.program_id(0)

    @pl.when(i < NP1)
    def _project():
        e = emb_ref[...].reshape(B * PB1, F_IN)
        t_all = jnp.dot(e, w_emb_ref[...], preferred_element_type=jnp.float32)
        rw = jnp.dot(refc_ref[...], w_ref_ref[...], preferred_element_type=jnp.float32)
        row = i * PB1
        for b in range(B):
            tpk_scr[pl.ds(row, PB1), b * G:b * G + NCOL] = (
                t_all[b * PB1:(b + 1) * PB1, :] + rw
            )

    @pl.when(i >= NP1)
    def _aggregate():
        acc = jnp.dot(adj_ref[...], tpk_scr[...], preferred_element_type=jnp.float32)
        tp = tpk_scr[pl.ds((i - NP1) * PBLK, PBLK), :]
        bz = bias_ref[...]
        for b in range(B):
            g = b * G
            pm_ref[b] = jnp.tanh(acc[:, g:g + 3] + tp[:, g + 6:g + 9] + bz[:, g:g + 3])
            rgb_ref[b] = jax.nn.sigmoid(
                acc[:, g + 3:g + 6] + tp[:, g + 9:g + 12] + bz[:, g + 3:g + 6]
            )


def kernel(embeddings, ref, adj, W_d, Wl_d, b_d, W_r, Wl_r, b_r):
    refc = ref.reshape(P, 3)
    W_all = jnp.concatenate([W_d, W_r, Wl_d, Wl_r], axis=1)
    W_emb = W_all[:F_IN]
    W_ref = W_all[F_IN:]
    bias = jnp.tile(
        jnp.concatenate([b_d, b_r, jnp.zeros((G - 6,), jnp.float32)]), B
    ).reshape(1, NP)

    pm, rgb = pl.pallas_call(
        _fused_kernel,
        grid=(NP1 + NPB,),
        in_specs=[
            pl.BlockSpec((B, PB1, F_IN),
                         lambda i: (0, jnp.minimum(i, NP1 - 1), 0)),
            pl.BlockSpec((PB1, 3), lambda i: (jnp.minimum(i, NP1 - 1), 0)),
            pl.BlockSpec((F_IN, NCOL), lambda i: (0, 0)),
            pl.BlockSpec((3, NCOL), lambda i: (0, 0)),
            pl.BlockSpec((PBLK, P),
                         lambda i: (jnp.clip(i - NP1, 0, NPB - 1), 0)),
            pl.BlockSpec((1, NP), lambda i: (0, 0)),
        ],
        out_specs=[
            pl.BlockSpec((B, PBLK, 3), lambda i: (0, jnp.maximum(i - NP1, 0), 0)),
            pl.BlockSpec((B, PBLK, 3), lambda i: (0, jnp.maximum(i - NP1, 0), 0)),
        ],
        out_shape=[
            jax.ShapeDtypeStruct((B, P, 3), jnp.float32),
            jax.ShapeDtypeStruct((B, P, 3), jnp.float32),
        ],
        scratch_shapes=[pltpu.VMEM((P, NP), jnp.float32)],
        compiler_params=pltpu.CompilerParams(
            dimension_semantics=("arbitrary",),
        ),
    )(embeddings, refc, W_emb, W_ref, adj, bias)
    return pm, rgb
